# Initial kernel scaffold; baseline (speedup 1.0000x reference)
import jax
import jax.numpy as jnp
from jax.experimental import pallas as pl


def kernel(points, W1t, b1t, W2t, b2t, W1p, b1p, W2p, b2p):
    raise NotImplementedError("write your pallas kernel here")


# speedup vs baseline: 2.2605x; 2.2605x over previous
import functools
import math

import jax
import jax.numpy as jnp
from jax.experimental import pallas as pl


def _mlp_pair_kernel(x_ref, w1c_ref, b1c_ref, w2t_ref, b2t_ref,
                     w2p_ref, b2p_ref, out_t_ref, out_p_ref):
    x = x_ref[...]
    h = jnp.dot(x, w1c_ref[...], preferred_element_type=jnp.float32)
    h = h + b1c_ref[...]
    g = h * (0.5 * (1.0 + jax.lax.erf(h * (1.0 / math.sqrt(2.0)))))
    y_t = jnp.dot(g[:, :64], w2t_ref[...], preferred_element_type=jnp.float32)
    out_t_ref[...] = y_t + b2t_ref[...]
    y_p = jnp.dot(g[:, 64:], w2p_ref[...], preferred_element_type=jnp.float32)
    out_p_ref[...] = y_p + b2p_ref[...]


@functools.partial(jax.jit, static_argnames=())
def kernel(points, W1t, b1t, W2t, b2t, W1p, b1p, W2p, b2p):
    B, N, C = points.shape
    D = W2t.shape[0]
    H = W1t.shape[0]
    BN = B * N

    x = points.reshape(BN, C)
    w1c = jnp.concatenate([W1t.T, W1p.T], axis=1)
    b1c = jnp.concatenate([b1t, b1p])[None, :]
    w2t = W2t.T
    w2p = W2p.T
    b2t2 = b2t[None, :]
    b2p2 = b2p[None, :]

    BLK = 4096
    grid = (BN // BLK,)

    out_t, out_p = pl.pallas_call(
        _mlp_pair_kernel,
        grid=grid,
        in_specs=[
            pl.BlockSpec((BLK, C), lambda i: (i, 0)),
            pl.BlockSpec((C, 2 * H), lambda i: (0, 0)),
            pl.BlockSpec((1, 2 * H), lambda i: (0, 0)),
            pl.BlockSpec((H, D), lambda i: (0, 0)),
            pl.BlockSpec((1, D), lambda i: (0, 0)),
            pl.BlockSpec((H, D), lambda i: (0, 0)),
            pl.BlockSpec((1, D), lambda i: (0, 0)),
        ],
        out_specs=[
            pl.BlockSpec((BLK, D), lambda i: (i, 0)),
            pl.BlockSpec((BLK, D), lambda i: (i, 0)),
        ],
        out_shape=[
            jax.ShapeDtypeStruct((BN, D), jnp.float32),
            jax.ShapeDtypeStruct((BN, D), jnp.float32),
        ],
    )(x, w1c, b1c, w2t, b2t2, w2p, b2p2)

    return (out_t.reshape(B, N, D), out_p.reshape(B, N, D))


# speedup vs baseline: 3.6178x; 1.6004x over previous
import functools
import math

import jax
import jax.numpy as jnp
from jax.experimental import pallas as pl


def _make_body(B):
    def body(x_ref, w1c_ref, b1c_ref, w2t_ref, b2t_ref,
             w2p_ref, b2p_ref, out_t_ref, out_p_ref):
        C = x_ref.shape[0]
        blk = x_ref.shape[-1]
        x = x_ref[...].reshape(C * B, blk)
        w1c = w1c_ref[...]
        b1c = b1c_ref[...]
        w2t = w2t_ref[...]
        b2t = b2t_ref[...]
        w2p = w2p_ref[...]
        b2p = b2p_ref[...]
        H = w2t.shape[0]
        for b in range(B):
            xb = jnp.concatenate([x[c * B + b:c * B + b + 1, :]
                                  for c in range(C)], axis=0)
            h = jax.lax.dot_general(xb, w1c, (((0,), (0,)), ((), ())),
                                    preferred_element_type=jnp.float32)
            h = h + b1c
            g = h * (0.5 * (1.0 + jax.lax.erf(h * (1.0 / math.sqrt(2.0)))))
            out_t_ref[b] = jnp.dot(g[:, :H], w2t,
                                   preferred_element_type=jnp.float32) + b2t
            out_p_ref[b] = jnp.dot(g[:, H:], w2p,
                                   preferred_element_type=jnp.float32) + b2p
    return body


@functools.partial(jax.jit, static_argnames=())
def kernel(points, W1t, b1t, W2t, b2t, W1p, b1p, W2p, b2p):
    B, N, C = points.shape
    D = W2t.shape[0]
    H = W1t.shape[0]

    xT = jnp.transpose(points, (2, 0, 1))

    w1c = jnp.concatenate([W1t.T, W1p.T], axis=1)
    b1c = jnp.concatenate([b1t, b1p])[None, :]
    w2t = W2t.T
    w2p = W2p.T
    b2t2 = b2t[None, :]
    b2p2 = b2p[None, :]

    BLK = 512
    grid = (N // BLK,)

    out_t, out_p = pl.pallas_call(
        _make_body(B),
        grid=grid,
        in_specs=[
            pl.BlockSpec((C, B, BLK), lambda i: (0, 0, i)),
            pl.BlockSpec((C, 2 * H), lambda i: (0, 0)),
            pl.BlockSpec((1, 2 * H), lambda i: (0, 0)),
            pl.BlockSpec((H, D), lambda i: (0, 0)),
            pl.BlockSpec((1, D), lambda i: (0, 0)),
            pl.BlockSpec((H, D), lambda i: (0, 0)),
            pl.BlockSpec((1, D), lambda i: (0, 0)),
        ],
        out_specs=[
            pl.BlockSpec((B, BLK, D), lambda i: (0, i, 0)),
            pl.BlockSpec((B, BLK, D), lambda i: (0, i, 0)),
        ],
        out_shape=[
            jax.ShapeDtypeStruct((B, N, D), jnp.float32),
            jax.ShapeDtypeStruct((B, N, D), jnp.float32),
        ],
    )(xT, w1c, b1c, w2t, b2t2, w2p, b2p2)

    return (out_t, out_p)


# speedup vs baseline: 4.3567x; 1.2043x over previous
import functools
import math

import jax
import jax.numpy as jnp
from jax.experimental import pallas as pl


def _make_body(B, BLK):
    def body(x_ref, w1_ref, b1_ref, w2_ref, b2_ref, out_t_ref, out_p_ref):
        i = pl.program_id(0)
        CB = x_ref.shape[0] * x_ref.shape[1]
        xb = x_ref[:, :, pl.ds(i * BLK, BLK)].reshape(CB, BLK)
        h = jax.lax.dot_general(xb, w1_ref[...], (((0,), (0,)), ((), ())),
                                preferred_element_type=jnp.float32)
        h = h + b1_ref[...]
        g = h * (0.5 * (1.0 + jax.lax.erf(h * (1.0 / math.sqrt(2.0)))))
        D = out_t_ref.shape[-1]
        w2 = w2_ref[...]
        b2 = b2_ref[...]
        for b in range(B):
            y = jnp.dot(g[:, b * D:(b + 1) * D], w2,
                        preferred_element_type=jnp.float32) + b2
            out_t_ref[b] = y[:, :D]
            out_p_ref[b] = y[:, D:]
    return body


@functools.partial(jax.jit, static_argnames=())
def kernel(points, W1t, b1t, W2t, b2t, W1p, b1p, W2p, b2p):
    B, N, C = points.shape
    D = W2t.shape[0]
    H = W1t.shape[0]

    xT = jnp.transpose(points, (2, 0, 1))

    w1c = jnp.concatenate([W1t.T, W1p.T], axis=1)
    eye_b = jnp.eye(B, dtype=jnp.float32)
    w1big = (w1c[:, None, None, :] * eye_b[None, :, :, None]
             ).reshape(C * B, B * 2 * H)
    b1c = jnp.concatenate([b1t, b1p])
    b1big = jnp.tile(b1c, B)[None, :]

    w2c = jnp.zeros((2 * H, 2 * D), jnp.float32)
    w2c = w2c.at[:H, :D].set(W2t.T).at[H:, D:].set(W2p.T)
    b2c = jnp.concatenate([b2t, b2p])[None, :]

    BLK = 512
    grid = (N // BLK,)

    out_t, out_p = pl.pallas_call(
        _make_body(B, BLK),
        grid=grid,
        in_specs=[
            pl.BlockSpec((C, B, N), lambda i: (0, 0, 0)),
            pl.BlockSpec((C * B, B * 2 * H), lambda i: (0, 0)),
            pl.BlockSpec((1, B * 2 * H), lambda i: (0, 0)),
            pl.BlockSpec((2 * H, 2 * D), lambda i: (0, 0)),
            pl.BlockSpec((1, 2 * D), lambda i: (0, 0)),
        ],
        out_specs=[
            pl.BlockSpec((B, BLK, D), lambda i: (0, i, 0)),
            pl.BlockSpec((B, BLK, D), lambda i: (0, i, 0)),
        ],
        out_shape=[
            jax.ShapeDtypeStruct((B, N, D), jnp.float32),
            jax.ShapeDtypeStruct((B, N, D), jnp.float32),
        ],
    )(xT, w1big, b1big, w2c, b2c)

    return (out_t, out_p)


# speedup vs baseline: 4.4900x; 1.0306x over previous
import functools
import math

import jax
import jax.numpy as jnp
from jax.experimental import pallas as pl


def _make_body(B, BLK):
    def body(x_ref, w1_ref, b1_ref, w2_ref, b2_ref, out_t_ref, out_p_ref):
        i = pl.program_id(0)
        CB = x_ref.shape[0] * x_ref.shape[1]
        xb = x_ref[:, :, pl.ds(i * BLK, BLK)].reshape(CB, BLK)
        h = jax.lax.dot_general(xb, w1_ref[...], (((0,), (0,)), ((), ())),
                                preferred_element_type=jnp.float32)
        h = h + b1_ref[...]
        g = h * (1.0 + jax.lax.erf(h * (1.0 / math.sqrt(2.0))))
        D = out_t_ref.shape[-1]
        w2 = w2_ref[...]
        b2 = b2_ref[...]
        for b in range(B):
            y = jnp.dot(g[:, b * D:(b + 1) * D], w2,
                        preferred_element_type=jnp.float32) + b2
            out_t_ref[b] = y[:, :D]
            out_p_ref[b] = y[:, D:]
    return body


@functools.partial(jax.jit, static_argnames=())
def kernel(points, W1t, b1t, W2t, b2t, W1p, b1p, W2p, b2p):
    B, N, C = points.shape
    D = W2t.shape[0]
    H = W1t.shape[0]

    xT = jnp.transpose(points, (2, 0, 1))

    w1c = jnp.concatenate([W1t.T, W1p.T], axis=1)
    eye_b = jnp.eye(B, dtype=jnp.float32)
    w1big = (w1c[:, None, None, :] * eye_b[None, :, :, None]
             ).reshape(C * B, B * 2 * H)
    b1c = jnp.concatenate([b1t, b1p])
    b1big = jnp.tile(b1c, B)[None, :]

    w2c = jnp.zeros((2 * H, 2 * D), jnp.float32)
    w2c = w2c.at[:H, :D].set(0.5 * W2t.T).at[H:, D:].set(0.5 * W2p.T)
    b2c = jnp.concatenate([b2t, b2p])[None, :]

    BLK = 1024
    grid = (N // BLK,)

    out_t, out_p = pl.pallas_call(
        _make_body(B, BLK),
        grid=grid,
        in_specs=[
            pl.BlockSpec((C, B, N), lambda i: (0, 0, 0)),
            pl.BlockSpec((C * B, B * 2 * H), lambda i: (0, 0)),
            pl.BlockSpec((1, B * 2 * H), lambda i: (0, 0)),
            pl.BlockSpec((2 * H, 2 * D), lambda i: (0, 0)),
            pl.BlockSpec((1, 2 * D), lambda i: (0, 0)),
        ],
        out_specs=[
            pl.BlockSpec((B, BLK, D), lambda i: (0, i, 0)),
            pl.BlockSpec((B, BLK, D), lambda i: (0, i, 0)),
        ],
        out_shape=[
            jax.ShapeDtypeStruct((B, N, D), jnp.float32),
            jax.ShapeDtypeStruct((B, N, D), jnp.float32),
        ],
    )(xT, w1big, b1big, w2c, b2c)

    return (out_t, out_p)
